# baseline (speedup 1.0000x reference)
import numpy as np
import jax
import jax.numpy as jnp
from jax.experimental import pallas as pl

_K = 512
_N_CW = 1024
_N_MOTHER = 1024
_CRC_LEN = 11
_NB_STAGES = 10

_CRC11_POLY = np.array([1, 1, 1, 0, 0, 0, 1, 0, 0, 0, 0, 1], dtype=np.int64)


def _crc_gen_matrix(k, poly_bits):
    crc_len = len(poly_bits) - 1
    G = np.zeros((k, crc_len), dtype=np.int64)
    for i in range(k):
        msg = np.zeros(k + crc_len, dtype=np.int64)
        msg[i] = 1
        for j in range(k):
            if msg[j]:
                msg[j:j + crc_len + 1] ^= poly_bits
        G[i] = msg[k:]
    return G


def _pw_info_pos(n_mother, k_total):
    beta = 2.0 ** 0.25
    nb = int(np.log2(n_mother))
    w = np.zeros(n_mother)
    for i in range(n_mother):
        for j in range(nb):
            if (i >> j) & 1:
                w[i] += beta ** j
    order = np.argsort(w)
    return np.sort(order[-k_total:])


def _gen_ind_gather(n):
    nb = int(np.log2(n))
    ind = np.ones((nb, n + 1), dtype=np.int64) * n
    for s in range(nb):
        r = np.arange(n // 2)
        d = r * 2 - np.mod(r, 2 ** s)
        ind[s, d] = d + 2 ** s
    return ind


def _subblock_perm(n):
    P = np.array([0, 1, 2, 4, 3, 5, 6, 7, 8, 16, 9, 17, 10, 18, 11, 19, 12,
                  20, 13, 21, 14, 22, 15, 23, 25, 26, 28, 31, 27, 29, 30, 24],
                 dtype=np.int64)
    per = n // 32
    idx = np.arange(n)
    return P[idx // per] * per + idx % per


def _triangular_perm(e):
    T = 1
    while T * (T + 1) // 2 < e:
        T += 1
    v = -np.ones((T, T), dtype=np.int64)
    c = 0
    for i in range(T):
        for j in range(T - i):
            if c < e:
                v[i, j] = c
                c += 1
    out = []
    for j in range(T):
        for i in range(T):
            if v[i, j] >= 0:
                out.append(v[i, j])
    return np.array(out, dtype=np.int64)


def _build_encoder_matrix():
    G_crc = (_crc_gen_matrix(_K, _CRC11_POLY) % 2).astype(np.uint8)
    info_pos = _pw_info_pos(_N_MOTHER, _K + _CRC_LEN)
    ind = _gen_ind_gather(_N_MOTHER)
    j_sb = _subblock_perm(_N_MOTHER)
    j_tri = _triangular_perm(_N_CW)

    u_crc = np.concatenate([np.eye(_K, dtype=np.uint8), G_crc], axis=1)
    c = np.zeros((_K, _N_MOTHER), dtype=np.uint8)
    c[:, info_pos] = u_crc
    x = np.concatenate([c, np.zeros((_K, 1), dtype=np.uint8)], axis=1)
    for s in range(_NB_STAGES):
        x = x ^ x[:, ind[s]]
    cw = x[:, :_N_MOTHER]
    return cw[:, j_sb][:, j_tri]


_M = jnp.asarray(_build_encoder_matrix(), dtype=jnp.bfloat16)


def _polar_mm_kernel(a_ref, m_ref, o_ref):
    a = a_ref[...].astype(jnp.bfloat16)
    acc = jnp.dot(a, m_ref[...], preferred_element_type=jnp.float32)
    o_ref[...] = acc - 2.0 * jnp.floor(acc * 0.5)


def kernel(inputs):
    b = inputs.shape[0]
    tb = 4096
    return pl.pallas_call(
        _polar_mm_kernel,
        grid=(b // tb,),
        in_specs=[
            pl.BlockSpec((tb, _K), lambda i: (i, 0)),
            pl.BlockSpec((_K, _N_CW), lambda i: (0, 0)),
        ],
        out_specs=pl.BlockSpec((tb, _N_CW), lambda i: (i, 0)),
        out_shape=jax.ShapeDtypeStruct((b, _N_CW), jnp.float32),
    )(inputs, _M)


# speedup vs baseline: 1.2106x; 1.2106x over previous
import numpy as np
import jax
import jax.numpy as jnp
from jax.experimental import pallas as pl
from jax.experimental.pallas import tpu as pltpu

_K = 512
_N_CW = 1024
_N_MOTHER = 1024
_CRC_LEN = 11
_NB_STAGES = 10

_CRC11_POLY = np.array([1, 1, 1, 0, 0, 0, 1, 0, 0, 0, 0, 1], dtype=np.int64)


def _crc_gen_matrix(k, poly_bits):
    crc_len = len(poly_bits) - 1
    G = np.zeros((k, crc_len), dtype=np.int64)
    for i in range(k):
        msg = np.zeros(k + crc_len, dtype=np.int64)
        msg[i] = 1
        for j in range(k):
            if msg[j]:
                msg[j:j + crc_len + 1] ^= poly_bits
        G[i] = msg[k:]
    return G


def _pw_info_pos(n_mother, k_total):
    beta = 2.0 ** 0.25
    nb = int(np.log2(n_mother))
    w = np.zeros(n_mother)
    for i in range(n_mother):
        for j in range(nb):
            if (i >> j) & 1:
                w[i] += beta ** j
    order = np.argsort(w)
    return np.sort(order[-k_total:])


def _gen_ind_gather(n):
    nb = int(np.log2(n))
    ind = np.ones((nb, n + 1), dtype=np.int64) * n
    for s in range(nb):
        r = np.arange(n // 2)
        d = r * 2 - np.mod(r, 2 ** s)
        ind[s, d] = d + 2 ** s
    return ind


def _subblock_perm(n):
    P = np.array([0, 1, 2, 4, 3, 5, 6, 7, 8, 16, 9, 17, 10, 18, 11, 19, 12,
                  20, 13, 21, 14, 22, 15, 23, 25, 26, 28, 31, 27, 29, 30, 24],
                 dtype=np.int64)
    per = n // 32
    idx = np.arange(n)
    return P[idx // per] * per + idx % per


def _triangular_perm(e):
    T = 1
    while T * (T + 1) // 2 < e:
        T += 1
    v = -np.ones((T, T), dtype=np.int64)
    c = 0
    for i in range(T):
        for j in range(T - i):
            if c < e:
                v[i, j] = c
                c += 1
    out = []
    for j in range(T):
        for i in range(T):
            if v[i, j] >= 0:
                out.append(v[i, j])
    return np.array(out, dtype=np.int64)


def _build_encoder_matrix():
    G_crc = (_crc_gen_matrix(_K, _CRC11_POLY) % 2).astype(np.uint8)
    info_pos = _pw_info_pos(_N_MOTHER, _K + _CRC_LEN)
    ind = _gen_ind_gather(_N_MOTHER)
    j_sb = _subblock_perm(_N_MOTHER)
    j_tri = _triangular_perm(_N_CW)

    u_crc = np.concatenate([np.eye(_K, dtype=np.uint8), G_crc], axis=1)
    c = np.zeros((_K, _N_MOTHER), dtype=np.uint8)
    c[:, info_pos] = u_crc
    x = np.concatenate([c, np.zeros((_K, 1), dtype=np.uint8)], axis=1)
    for s in range(_NB_STAGES):
        x = x ^ x[:, ind[s]]
    cw = x[:, :_N_MOTHER]
    return cw[:, j_sb][:, j_tri]


_M = jnp.asarray(_build_encoder_matrix(), dtype=jnp.bfloat16)


def _polar_mm_kernel(a_ref, m_ref, o_ref):
    a = a_ref[...].astype(jnp.bfloat16)
    acc = jnp.dot(a, m_ref[...], preferred_element_type=jnp.float32)
    o_ref[...] = acc - 2.0 * jnp.floor(acc * 0.5)


def kernel(inputs):
    b = inputs.shape[0]
    tb = 1024
    return pl.pallas_call(
        _polar_mm_kernel,
        grid=(b // tb,),
        compiler_params=pltpu.CompilerParams(
            dimension_semantics=("parallel",)),
        in_specs=[
            pl.BlockSpec((tb, _K), lambda i: (i, 0)),
            pl.BlockSpec((_K, _N_CW), lambda i: (0, 0)),
        ],
        out_specs=pl.BlockSpec((tb, _N_CW), lambda i: (i, 0)),
        out_shape=jax.ShapeDtypeStruct((b, _N_CW), jnp.float32),
    )(inputs, _M)


# speedup vs baseline: 1.2869x; 1.0630x over previous
import numpy as np
import jax
import jax.numpy as jnp
from jax.experimental import pallas as pl
from jax.experimental.pallas import tpu as pltpu

_K = 512
_N_CW = 1024
_N_MOTHER = 1024
_CRC_LEN = 11
_NB_STAGES = 10

_CRC11_POLY = np.array([1, 1, 1, 0, 0, 0, 1, 0, 0, 0, 0, 1], dtype=np.int64)


def _crc_gen_matrix(k, poly_bits):
    crc_len = len(poly_bits) - 1
    G = np.zeros((k, crc_len), dtype=np.int64)
    for i in range(k):
        msg = np.zeros(k + crc_len, dtype=np.int64)
        msg[i] = 1
        for j in range(k):
            if msg[j]:
                msg[j:j + crc_len + 1] ^= poly_bits
        G[i] = msg[k:]
    return G


def _pw_info_pos(n_mother, k_total):
    beta = 2.0 ** 0.25
    nb = int(np.log2(n_mother))
    w = np.zeros(n_mother)
    for i in range(n_mother):
        for j in range(nb):
            if (i >> j) & 1:
                w[i] += beta ** j
    order = np.argsort(w)
    return np.sort(order[-k_total:])


def _gen_ind_gather(n):
    nb = int(np.log2(n))
    ind = np.ones((nb, n + 1), dtype=np.int64) * n
    for s in range(nb):
        r = np.arange(n // 2)
        d = r * 2 - np.mod(r, 2 ** s)
        ind[s, d] = d + 2 ** s
    return ind


def _subblock_perm(n):
    P = np.array([0, 1, 2, 4, 3, 5, 6, 7, 8, 16, 9, 17, 10, 18, 11, 19, 12,
                  20, 13, 21, 14, 22, 15, 23, 25, 26, 28, 31, 27, 29, 30, 24],
                 dtype=np.int64)
    per = n // 32
    idx = np.arange(n)
    return P[idx // per] * per + idx % per


def _triangular_perm(e):
    T = 1
    while T * (T + 1) // 2 < e:
        T += 1
    v = -np.ones((T, T), dtype=np.int64)
    c = 0
    for i in range(T):
        for j in range(T - i):
            if c < e:
                v[i, j] = c
                c += 1
    out = []
    for j in range(T):
        for i in range(T):
            if v[i, j] >= 0:
                out.append(v[i, j])
    return np.array(out, dtype=np.int64)


def _build_encoder_matrix():
    G_crc = (_crc_gen_matrix(_K, _CRC11_POLY) % 2).astype(np.uint8)
    info_pos = _pw_info_pos(_N_MOTHER, _K + _CRC_LEN)
    ind = _gen_ind_gather(_N_MOTHER)
    j_sb = _subblock_perm(_N_MOTHER)
    j_tri = _triangular_perm(_N_CW)

    u_crc = np.concatenate([np.eye(_K, dtype=np.uint8), G_crc], axis=1)
    c = np.zeros((_K, _N_MOTHER), dtype=np.uint8)
    c[:, info_pos] = u_crc
    x = np.concatenate([c, np.zeros((_K, 1), dtype=np.uint8)], axis=1)
    for s in range(_NB_STAGES):
        x = x ^ x[:, ind[s]]
    cw = x[:, :_N_MOTHER]
    return cw[:, j_sb][:, j_tri]


_M_NP = _build_encoder_matrix().astype(np.float32)


def _polar_mm_kernel(a_ref, m_ref, o_ref):
    a = a_ref[...].astype(jnp.bfloat16)
    acc = jnp.dot(a, m_ref[...], preferred_element_type=jnp.float32)
    o_ref[...] = acc - 2.0 * jnp.floor(acc * 0.5)


def kernel(inputs):
    b = inputs.shape[0]
    tb = 2048
    return pl.pallas_call(
        _polar_mm_kernel,
        grid=(b // tb,),
        compiler_params=pltpu.CompilerParams(
            dimension_semantics=("arbitrary",)),
        in_specs=[
            pl.BlockSpec((tb, _K), lambda i: (i, 0)),
            pl.BlockSpec((_K, _N_CW), lambda i: (0, 0)),
        ],
        out_specs=pl.BlockSpec((tb, _N_CW), lambda i: (i, 0)),
        out_shape=jax.ShapeDtypeStruct((b, _N_CW), jnp.float32),
    )(inputs, jnp.asarray(_M_NP, dtype=jnp.bfloat16))
